# Initial kernel scaffold; baseline (speedup 1.0000x reference)
import jax
import jax.numpy as jnp
from jax.experimental import pallas as pl


def kernel(embeddings, labels, query_embeddings):
    raise NotImplementedError("write your pallas kernel here")


# speedup vs baseline: 2.7455x; 2.7455x over previous
import functools

import jax
import jax.numpy as jnp
from jax import lax
from jax.experimental import pallas as pl
from jax.experimental.pallas import tpu as pltpu
from jax.experimental.pallas import tpu_sc as plsc

NUM_CLASSES = 1000
N_SUPPORT = 1000000
N_QUERY = 16384
DIM = 64

NC = 2
NS = 16
NW = NC * NS

CHUNK = 384
EWORDS = CHUNK * DIM
NCHUNKS = N_SUPPORT // CHUNK
BASE_CHUNKS = NCHUNKS // NW
EXTRA = NCHUNKS - BASE_CHUNKS * NW
TAIL_ROW = NCHUNKS * CHUNK
TAIL_N = N_SUPPORT - TAIL_ROW

CPAD = 1008
AWORDS = CPAD * DIM
CNTPAD = 1024
CWORDS = CNTPAD * 16
SWIN = 4096
CWIN = 1024


def _sc_segment_sums(emb1d, lab1d):
  mesh = plsc.VectorSubcoreMesh(core_axis_name="c", subcore_axis_name="s")

  @functools.partial(
      pl.kernel,
      out_type=(
          jax.ShapeDtypeStruct((NC * AWORDS,), jnp.float32),
          jax.ShapeDtypeStruct((NC * CWORDS,), jnp.float32),
      ),
      mesh=mesh,
      scratch_types=[
          pltpu.VMEM((EWORDS,), jnp.float32),
          pltpu.VMEM((CHUNK,), jnp.int32),
          pltpu.VMEM((AWORDS,), jnp.float32),
          pltpu.VMEM((CWORDS,), jnp.float32),
          pltpu.VMEM_SHARED((AWORDS,), jnp.float32),
          pltpu.VMEM_SHARED((CWORDS,), jnp.float32),
      ],
  )
  def seg(emb_hbm, lab_hbm, sums_out, counts_out, ebuf, lbuf, acc, cnt,
          ssum, scnt):
    cid = lax.axis_index("c")
    sid = lax.axis_index("s")
    wid = sid * NC + cid

    zeros16 = jnp.zeros((16,), jnp.float32)
    ones16 = jnp.ones((16,), jnp.float32)

    def zero_acc(i, _):
      acc[pl.ds(i * 16, 16)] = zeros16
      return 0
    lax.fori_loop(0, AWORDS // 16, zero_acc, 0)

    def zero_cnt(i, _):
      cnt[pl.ds(i * 16, 16)] = zeros16
      return 0
    lax.fori_loop(0, CWORDS // 16, zero_cnt, 0)

    def accum_group(g, _):
      lv = lbuf[pl.ds(g * 16, 16)]
      for k in range(16):
        l = lv[k]
        rbase = (g * 16 + k) * DIM
        abase = l * DIM
        for d in range(4):
          v = ebuf[pl.ds(rbase + d * 16, 16)]
          plsc.addupdate(acc.at[pl.ds(abase + d * 16, 16)], v)
        plsc.addupdate(cnt.at[pl.ds(l * 16, 16)], ones16)
      return 0

    nchunks_w = jnp.where(wid < EXTRA, BASE_CHUNKS + 1, BASE_CHUNKS)

    def chunk_body(i, _):
      c = wid + i * NW
      pltpu.sync_copy(lab_hbm.at[pl.ds(c * CHUNK, CHUNK)], lbuf)
      pltpu.sync_copy(emb_hbm.at[pl.ds(c * EWORDS, EWORDS)], ebuf)
      lax.fori_loop(0, CHUNK // 16, accum_group, 0)
      return 0

    lax.fori_loop(0, nchunks_w, chunk_body, 0)

    @pl.when(wid == NW - 1)
    def _():
      pltpu.sync_copy(lab_hbm.at[pl.ds(TAIL_ROW, TAIL_N)],
                      lbuf.at[pl.ds(0, TAIL_N)])
      pltpu.sync_copy(emb_hbm.at[pl.ds(TAIL_ROW * DIM, TAIL_N * DIM)],
                      ebuf.at[pl.ds(0, TAIL_N * DIM)])
      lax.fori_loop(0, TAIL_N // 16, accum_group, 0)

    plsc.subcore_barrier()

    sw0 = jnp.where(sid == 15, (AWORDS // SWIN - 1) * SWIN, sid * SWIN)
    cw0 = sid * CWIN

    SACC = SWIN
    CRD = 2 * SWIN
    CACC = 2 * SWIN + CWIN

    def zero2(i, _):
      ebuf[pl.ds(SACC + i * 16, 16)] = zeros16
      return 0
    lax.fori_loop(0, SWIN // 16, zero2, 0)

    def zero2c(i, _):
      ebuf[pl.ds(CACC + i * 16, 16)] = zeros16
      return 0
    lax.fori_loop(0, CWIN // 16, zero2c, 0)

    def round_body(k, _):
      @pl.when(sid == k)
      def _():
        pltpu.sync_copy(acc, ssum)
        pltpu.sync_copy(cnt, scnt)
      plsc.subcore_barrier()
      pltpu.sync_copy(ssum.at[pl.ds(sw0, SWIN)], ebuf.at[pl.ds(0, SWIN)])
      pltpu.sync_copy(scnt.at[pl.ds(cw0, CWIN)], ebuf.at[pl.ds(CRD, CWIN)])

      def vec_add(i, _):
        plsc.addupdate(ebuf.at[pl.ds(SACC + i * 16, 16)],
                       ebuf[pl.ds(i * 16, 16)])
        return 0
      lax.fori_loop(0, SWIN // 16, vec_add, 0)

      def vec_addc(i, _):
        plsc.addupdate(ebuf.at[pl.ds(CACC + i * 16, 16)],
                       ebuf[pl.ds(CRD + i * 16, 16)])
        return 0
      lax.fori_loop(0, CWIN // 16, vec_addc, 0)
      plsc.subcore_barrier()
      return 0

    lax.fori_loop(0, NS, round_body, 0)

    pltpu.sync_copy(ebuf.at[pl.ds(SACC, SWIN)],
                    sums_out.at[pl.ds(cid * AWORDS + sw0, SWIN)])
    pltpu.sync_copy(ebuf.at[pl.ds(CACC, CWIN)],
                    counts_out.at[pl.ds(cid * CWORDS + cw0, CWIN)])

  return seg(emb1d, lab1d)


def _combine_body(sums_ref, counts_ref, proto_ref):
  s = sums_ref[0] + sums_ref[1]
  c = counts_ref[0] + counts_ref[1]
  proto_ref[...] = s[:NUM_CLASSES] / jnp.maximum(c[:NUM_CLASSES, 0:1], 1.0)


def _tc_combine(sums, counts):
  return pl.pallas_call(
      _combine_body,
      out_shape=jax.ShapeDtypeStruct((NUM_CLASSES, DIM), jnp.float32),
  )(sums, counts)


def _tc_body(proto_ref, q_ref, out_ref):
  proto = proto_ref[...]
  q = q_ref[...]
  q2 = jnp.sum(q * q, axis=1, keepdims=True)
  dn = (((1,), (1,)), ((), ()))
  p2 = lax.dot_general(jnp.ones((1, DIM), jnp.float32), proto * proto, dn,
                       precision=lax.Precision.HIGHEST,
                       preferred_element_type=jnp.float32)
  qp = lax.dot_general(q, proto, dn,
                       precision=lax.Precision.HIGHEST,
                       preferred_element_type=jnp.float32)
  d2 = (q2 + p2) - 2.0 * qp
  dist = jnp.sqrt(jnp.maximum(d2, 1e-12))
  m = jnp.min(dist, axis=1, keepdims=True)
  e = jnp.exp(m - dist)
  out_ref[...] = e / jnp.sum(e, axis=1, keepdims=True)


def _tc_cdist_softmax(proto, q):
  bq = 1024
  grid = (N_QUERY // bq,)
  return pl.pallas_call(
      _tc_body,
      grid=grid,
      in_specs=[
          pl.BlockSpec((NUM_CLASSES, DIM), lambda i: (0, 0)),
          pl.BlockSpec((bq, DIM), lambda i: (i, 0)),
      ],
      out_specs=pl.BlockSpec((bq, NUM_CLASSES), lambda i: (i, 0)),
      out_shape=jax.ShapeDtypeStruct((N_QUERY, NUM_CLASSES), jnp.float32),
  )(proto, q)


@jax.jit
def kernel(embeddings, labels, query_embeddings):
  emb1d = embeddings.reshape(N_SUPPORT * DIM)
  lab1d = labels.astype(jnp.int32).reshape(N_SUPPORT)
  sums1d, counts1d = _sc_segment_sums(emb1d, lab1d)
  sums = sums1d.reshape(NC, CPAD, DIM)
  counts = counts1d.reshape(NC, CNTPAD, 16)
  proto = _tc_combine(sums, counts)
  return _tc_cdist_softmax(proto, query_embeddings)


# speedup vs baseline: 3.1313x; 1.1405x over previous
import functools

import jax
import jax.numpy as jnp
from jax import lax
from jax.experimental import pallas as pl
from jax.experimental.pallas import tpu as pltpu
from jax.experimental.pallas import tpu_sc as plsc

NUM_CLASSES = 1000
N_SUPPORT = 1000000
N_QUERY = 16384
DIM = 64

NC = 2
NS = 16
NW = NC * NS

CHUNK = 256
NCHUNKS = N_SUPPORT // CHUNK
BASE_CHUNKS = NCHUNKS // NW
EXTRA = NCHUNKS - BASE_CHUNKS * NW
TAIL_ROW = NCHUNKS * CHUNK
TAIL_N = N_SUPPORT - TAIL_ROW

CPAD = 1008
AWORDS = CPAD * DIM
CNTPAD = 1024
CWORDS = CNTPAD * 16


def _sc_segment_sums(embeddings, lab1d):
  mesh = plsc.VectorSubcoreMesh(core_axis_name="c", subcore_axis_name="s")

  @functools.partial(
      pl.kernel,
      out_type=(
          jax.ShapeDtypeStruct((NW * AWORDS,), jnp.float32),
          jax.ShapeDtypeStruct((NW * CWORDS,), jnp.float32),
      ),
      mesh=mesh,
      scratch_types=[
          pltpu.VMEM((CHUNK, DIM), jnp.float32),
          pltpu.VMEM((CHUNK,), jnp.int32),
          pltpu.VMEM((AWORDS,), jnp.float32),
          pltpu.VMEM((CWORDS,), jnp.float32),
      ],
  )
  def seg(emb_hbm, lab_hbm, sums_out, counts_out, ebuf, lbuf, acc, cnt):
    cid = lax.axis_index("c")
    sid = lax.axis_index("s")
    wid = sid * NC + cid

    zeros16 = jnp.zeros((16,), jnp.float32)
    ones16 = jnp.ones((16,), jnp.float32)

    def zero_acc(i, _):
      acc[pl.ds(i * 16, 16)] = zeros16
      return 0
    lax.fori_loop(0, AWORDS // 16, zero_acc, 0)

    def zero_cnt(i, _):
      cnt[pl.ds(i * 16, 16)] = zeros16
      return 0
    lax.fori_loop(0, CWORDS // 16, zero_cnt, 0)

    def accum_group(g, _):
      lv = lbuf[pl.ds(g * 16, 16)]
      for k in range(16):
        l = lv[k]
        r = g * 16 + k
        abase = l * DIM
        for d in range(4):
          v = ebuf[r, pl.ds(d * 16, 16)]
          plsc.addupdate(acc.at[pl.ds(abase + d * 16, 16)], v)
        plsc.addupdate(cnt.at[pl.ds(l * 16, 16)], ones16)
      return 0

    nchunks_w = jnp.where(wid < EXTRA, BASE_CHUNKS + 1, BASE_CHUNKS)

    def chunk_body(i, _):
      c = wid + i * NW
      pltpu.sync_copy(lab_hbm.at[pl.ds(c * CHUNK, CHUNK)], lbuf)
      pltpu.sync_copy(emb_hbm.at[pl.ds(c * CHUNK, CHUNK)], ebuf)
      lax.fori_loop(0, CHUNK // 16, accum_group, 0)
      return 0

    lax.fori_loop(0, nchunks_w, chunk_body, 0)

    @pl.when(wid == NW - 1)
    def _():
      pltpu.sync_copy(lab_hbm.at[pl.ds(TAIL_ROW, TAIL_N)],
                      lbuf.at[pl.ds(0, TAIL_N)])
      pltpu.sync_copy(emb_hbm.at[pl.ds(TAIL_ROW, TAIL_N)],
                      ebuf.at[pl.ds(0, TAIL_N)])
      lax.fori_loop(0, TAIL_N // 16, accum_group, 0)

    pltpu.sync_copy(acc, sums_out.at[pl.ds(wid * AWORDS, AWORDS)])
    pltpu.sync_copy(cnt, counts_out.at[pl.ds(wid * CWORDS, CWORDS)])

  return seg(embeddings, lab1d)


def _combine_body(sums_ref, counts_ref, proto_ref):
  s = jnp.sum(sums_ref[...], axis=0)
  c = jnp.sum(counts_ref[...], axis=0)
  proto_ref[...] = s[:NUM_CLASSES] / jnp.maximum(c[:NUM_CLASSES, 0:1], 1.0)


def _tc_combine(sums, counts):
  return pl.pallas_call(
      _combine_body,
      out_shape=jax.ShapeDtypeStruct((NUM_CLASSES, DIM), jnp.float32),
  )(sums, counts)


def _tc_body(proto_ref, q_ref, out_ref):
  proto = proto_ref[...]
  q = q_ref[...]
  q2 = jnp.sum(q * q, axis=1, keepdims=True)
  dn = (((1,), (1,)), ((), ()))
  p2 = lax.dot_general(jnp.ones((1, DIM), jnp.float32), proto * proto, dn,
                       precision=lax.Precision.HIGHEST,
                       preferred_element_type=jnp.float32)
  qp = lax.dot_general(q, proto, dn,
                       precision=lax.Precision.HIGHEST,
                       preferred_element_type=jnp.float32)
  d2 = (q2 + p2) - 2.0 * qp
  dist = jnp.sqrt(jnp.maximum(d2, 1e-12))
  m = jnp.min(dist, axis=1, keepdims=True)
  e = jnp.exp(m - dist)
  out_ref[...] = e / jnp.sum(e, axis=1, keepdims=True)


def _tc_cdist_softmax(proto, q):
  bq = 1024
  grid = (N_QUERY // bq,)
  return pl.pallas_call(
      _tc_body,
      grid=grid,
      in_specs=[
          pl.BlockSpec((NUM_CLASSES, DIM), lambda i: (0, 0)),
          pl.BlockSpec((bq, DIM), lambda i: (i, 0)),
      ],
      out_specs=pl.BlockSpec((bq, NUM_CLASSES), lambda i: (i, 0)),
      out_shape=jax.ShapeDtypeStruct((N_QUERY, NUM_CLASSES), jnp.float32),
  )(proto, q)


@jax.jit
def kernel(embeddings, labels, query_embeddings):
  lab1d = labels.astype(jnp.int32).reshape(N_SUPPORT)
  sums1d, counts1d = _sc_segment_sums(embeddings, lab1d)
  sums = sums1d.reshape(NW, CPAD, DIM)
  counts = counts1d.reshape(NW, CNTPAD, 16)
  proto = _tc_combine(sums, counts)
  return _tc_cdist_softmax(proto, query_embeddings)
